# baseline (speedup 1.0000x reference)
import functools

import jax
import jax.numpy as jnp
from jax import lax
from jax.experimental import pallas as pl
from jax.experimental.pallas import tpu as pltpu
from jax.experimental.pallas import tpu_sc as plsc

NUM_FIELDS = 26
VOCAB = 100000
EMB_DIM = 32
BATCH = 16384

_INFO = plsc.get_sparse_core_info()
_NC, _NS, _L = _INFO.num_cores, _INFO.num_subcores, _INFO.num_lanes
_NW = _NC * _NS
_HALF = BATCH // 2
_NSPLIT = 2


def _sc_embedding_t(xt, tabt, f0, nf):
    mesh = plsc.VectorSubcoreMesh(core_axis_name="c", subcore_axis_name="s")

    @functools.partial(
        pl.kernel,
        mesh=mesh,
        out_type=jax.ShapeDtypeStruct((nf * EMB_DIM, BATCH), jnp.float32),
        scratch_types=[
            pltpu.VMEM((VOCAB,), jnp.float32),
            pltpu.VMEM((_HALF,), jnp.int32),
            pltpu.VMEM((2, _HALF), jnp.float32),
            pltpu.SemaphoreType.DMA,
            pltpu.SemaphoreType.DMA,
        ],
        compiler_params=pltpu.CompilerParams(use_tc_tiling_on_sc=False,
                                             needs_layout_passes=False),
    )
    def k(xt_hbm, tab_hbm, out_hbm, vocab_v, idx_v, outr_v, gsem, wsem):
        d = lax.axis_index("s") * _NC + lax.axis_index("c")

        def drain_write(slot):
            pltpu.make_async_copy(xt_hbm.at[0, pl.ds(0, _HALF)],
                                  outr_v.at[slot], wsem).wait()

        def fbody(f, carry):
            row = f * EMB_DIM + d
            pltpu.sync_copy(tab_hbm.at[row], vocab_v)
            for h in range(2):
                pltpu.sync_copy(xt_hbm.at[f0 + f, pl.ds(h * _HALF, _HALF)],
                                idx_v)

                @pl.when(f > 0)
                def _():
                    drain_write(h)

                def gbody(j, c):
                    sl = pl.ds(j * _L, _L)
                    iv = idx_v[sl]
                    outr_v[h, sl] = plsc.load_gather(vocab_v, [iv])
                    return c

                lax.fori_loop(0, _HALF // _L, gbody, 0)
                pltpu.async_copy(outr_v.at[h],
                                 out_hbm.at[row, pl.ds(h * _HALF, _HALF)],
                                 wsem)
            return carry

        lax.fori_loop(0, nf, fbody, 0)
        drain_write(0)
        drain_write(1)

    return k(xt, tabt)


def kernel(x_cat, tables):
    xt = x_cat.T.astype(jnp.int32)
    outs = []
    f0 = 0
    per = NUM_FIELDS // _NSPLIT
    for s in range(_NSPLIT):
        nf = per + (1 if s < NUM_FIELDS % _NSPLIT else 0)
        tabt = tables[f0:f0 + nf].transpose(0, 2, 1).reshape(
            nf * EMB_DIM, VOCAB)
        outs.append(_sc_embedding_t(xt, tabt, f0, nf))
        f0 += nf
    out_t = jnp.concatenate(outs, axis=0)
    return out_t.T.reshape(BATCH, NUM_FIELDS * EMB_DIM)
